# baseline (speedup 1.0000x reference)
import jax
import jax.numpy as jnp
from jax.experimental import pallas as pl
from jax.experimental.pallas import tpu as pltpu

_BLOCK_R = 512


def _body(mid_ref, x_ref, tab_ref, o_ref):
    o_ref[...] = x_ref[...] + tab_ref[0]


def kernel(x, mod_emb_table, modality_id):
    B, S, D = x.shape
    R = B * S
    M = mod_emb_table.shape[0]
    xf = x.reshape(R, D)
    tab3 = mod_emb_table.reshape(M, 1, D)
    mid = jnp.asarray(modality_id, jnp.int32).reshape(1)
    out = pl.pallas_call(
        _body,
        grid_spec=pltpu.PrefetchScalarGridSpec(
            num_scalar_prefetch=1,
            grid=(R // _BLOCK_R,),
            in_specs=[
                pl.BlockSpec((_BLOCK_R, D), lambda i, mid: (i, 0)),
                pl.BlockSpec((1, 1, D), lambda i, mid: (mid[0], 0, 0)),
            ],
            out_specs=pl.BlockSpec((_BLOCK_R, D), lambda i, mid: (i, 0)),
        ),
        out_shape=jax.ShapeDtypeStruct((R, D), x.dtype),
    )(mid, xf, tab3)
    return out.reshape(B, S, D)


# speedup vs baseline: 1.0195x; 1.0195x over previous
import jax
import jax.numpy as jnp
from jax.experimental import pallas as pl
from jax.experimental.pallas import tpu as pltpu

_BLOCK_R = 1024


def _body(mid_ref, x_ref, tab_ref, o_ref):
    o_ref[...] = x_ref[...] + tab_ref[0]


def kernel(x, mod_emb_table, modality_id):
    B, S, D = x.shape
    R = B * S
    M = mod_emb_table.shape[0]
    xf = x.reshape(R, D)
    tab3 = mod_emb_table.reshape(M, 1, D)
    mid = jnp.asarray(modality_id, jnp.int32).reshape(1)
    out = pl.pallas_call(
        _body,
        grid_spec=pltpu.PrefetchScalarGridSpec(
            num_scalar_prefetch=1,
            grid=(R // _BLOCK_R,),
            in_specs=[
                pl.BlockSpec((_BLOCK_R, D), lambda i, mid: (i, 0)),
                pl.BlockSpec((1, 1, D), lambda i, mid: (mid[0], 0, 0)),
            ],
            out_specs=pl.BlockSpec((_BLOCK_R, D), lambda i, mid: (i, 0)),
        ),
        out_shape=jax.ShapeDtypeStruct((R, D), x.dtype),
    )(mid, xf, tab3)
    return out.reshape(B, S, D)
